# baseline (speedup 1.0000x reference)
import functools

import jax
import jax.numpy as jnp
from jax import lax
from jax.experimental import pallas as pl
from jax.experimental.pallas import tpu as pltpu

D_E = 64
L_MAX = 2048
CHUNK = 32
NCH = L_MAX // CHUNK


def _emb_body(x_s, word_any, pos_v, out_v, gath_v, sems):
    def start_chunk(g, slot):
        base = g * CHUNK
        for j in range(CHUNK):
            r = base + j
            idx = x_s[r]
            pltpu.make_async_copy(
                word_any.at[pl.ds(idx, 1)], gath_v.at[pl.ds(r, 1)], sems.at[slot]
            ).start()

    def wait_chunk(g, slot):
        base = g * CHUNK
        for j in range(CHUNK):
            r = base + j
            idx = x_s[r]
            pltpu.make_async_copy(
                word_any.at[pl.ds(idx, 1)], gath_v.at[pl.ds(r, 1)], sems.at[slot]
            ).wait()

    start_chunk(0, 0)

    def loop_body(g, carry):
        @pl.when(g + 1 < NCH)
        def _():
            start_chunk(g + 1, (g + 1) % 2)

        wait_chunk(g, g % 2)
        return carry

    lax.fori_loop(0, NCH, loop_body, 0)
    out_v[...] = gath_v[...] + pos_v[...]


@functools.partial(jax.jit, static_argnames=())
def _emb(x, word_table, pos_table):
    return pl.pallas_call(
        _emb_body,
        grid=(),
        in_specs=[
            pl.BlockSpec(memory_space=pltpu.SMEM),
            pl.BlockSpec(memory_space=pltpu.HBM),
            pl.BlockSpec(memory_space=pltpu.VMEM),
        ],
        out_specs=pl.BlockSpec(memory_space=pltpu.VMEM),
        out_shape=jax.ShapeDtypeStruct((L_MAX, D_E), jnp.float32),
        scratch_shapes=[
            pltpu.VMEM((L_MAX, D_E), jnp.float32),
            pltpu.SemaphoreType.DMA((2,)),
        ],
    )(x, word_table, pos_table)


def kernel(x, word_table, pos_table):
    return _emb(x.astype(jnp.int32), word_table, pos_table)
